# baseline (speedup 1.0000x reference)
import functools
import jax
import jax.numpy as jnp
from jax import lax
from jax.experimental import pallas as pl
from jax.experimental.pallas import tpu as pltpu
from jax.experimental.pallas import tpu_sc as plsc

_T = 2048
_D = 1024
_E = 8
_K = 2
_F = 256
_BR = 256
_NB = (_K * _T) // _BR + (_E - 1)
_PR = _NB * _BR
_NW = 32
_TW = _T // _NW


def _shift_down(c, s):
    return jnp.concatenate([jnp.zeros((s,) + c.shape[1:], c.dtype), c[:-s]],
                           axis=0)


def _router_kernel(x_ref, gate_wt_ref, corr_ref,
                   xbf_ref, p_ref, w_ref, be_ref):
    bf = jnp.bfloat16
    x = x_ref[...].astype(bf)
    xbf_ref[...] = x

    logits = jnp.dot(x, gate_wt_ref[...].astype(bf),
                     preferred_element_type=jnp.float32)
    scores = jax.nn.softmax(logits, axis=-1)
    biased = scores + corr_ref[...]

    eidx = lax.broadcasted_iota(jnp.int32, (_T, _E), 1)
    m1 = jnp.max(biased, axis=-1, keepdims=True)
    i1 = jnp.min(jnp.where(biased == m1, eidx, _E), axis=-1, keepdims=True)
    b2 = jnp.where(eidx == i1, -jnp.inf, biased)
    m2 = jnp.max(b2, axis=-1, keepdims=True)
    i2 = jnp.min(jnp.where(b2 == m2, eidx, _E), axis=-1, keepdims=True)
    w1 = jnp.sum(jnp.where(eidx == i1, scores, 0.0), axis=-1, keepdims=True)
    w2 = jnp.sum(jnp.where(eidx == i2, scores, 0.0), axis=-1, keepdims=True)
    denom = w1 + w2
    w_ref[...] = jnp.concatenate([w1 / denom, w2 / denom], axis=1)

    oh = ((eidx == i1) | (eidx == i2)).astype(jnp.float32)
    csum = oh
    s = 1
    while s < _T:
        csum = csum + _shift_down(csum, s)
        s *= 2
    csum_ex = csum - oh
    counts = csum[_T - 1:_T, :]
    nb = jnp.floor((counts + (_BR - 1)) / _BR)
    acc = nb
    s = 1
    while s < _E:
        acc_s = jnp.concatenate(
            [jnp.zeros((1, s), acc.dtype), acc[:, :-s]], axis=1)
        acc = acc + acc_s
        s *= 2
    pof = acc - nb
    prow = pof * _BR

    pos_base = prow + csum_ex
    p1 = jnp.sum(jnp.where(eidx == i1, pos_base, 0.0), axis=-1, keepdims=True)
    p2 = jnp.sum(jnp.where(eidx == i2, pos_base, 0.0), axis=-1, keepdims=True)
    p_ref[...] = jnp.concatenate([p1, p2], axis=1).astype(jnp.int32)

    bidx = lax.broadcasted_iota(
        jnp.int32, (1, _NB + 9), 1).astype(jnp.float32)
    be = jnp.zeros((1, _NB + 9), jnp.float32)
    for e in range(_E):
        be = be + (bidx >= pof[:, e:e + 1]).astype(jnp.float32)
    be_ref[...] = jnp.clip(be - 1.0, 0.0, float(_E - 1)).astype(jnp.int32)


def _run_router(x, gate_wt, corr):
    return pl.pallas_call(
        _router_kernel,
        in_specs=[
            pl.BlockSpec((_T, _D), lambda: (0, 0)),
            pl.BlockSpec((_D, _E), lambda: (0, 0)),
            pl.BlockSpec((1, _E), lambda: (0, 0)),
        ],
        out_specs=[
            pl.BlockSpec((_T, _D), lambda: (0, 0)),
            pl.BlockSpec((_T, _K), lambda: (0, 0)),
            pl.BlockSpec((_T, _K), lambda: (0, 0)),
            pl.BlockSpec((1, _NB + 9), lambda: (0, 0)),
        ],
        out_shape=[
            jax.ShapeDtypeStruct((_T, _D), jnp.bfloat16),
            jax.ShapeDtypeStruct((_T, _K), jnp.int32),
            jax.ShapeDtypeStruct((_T, _K), jnp.float32),
            jax.ShapeDtypeStruct((1, _NB + 9), jnp.int32),
        ],
    )(x, gate_wt, corr)


def _dispatch(xi, p_sc):
    mesh = plsc.VectorSubcoreMesh(core_axis_name="c", subcore_axis_name="s")

    @functools.partial(
        pl.kernel, mesh=mesh,
        out_type=jax.ShapeDtypeStruct((_PR, _D), jnp.float32),
        scratch_types=[
            pltpu.VMEM((_TW,), jnp.int32),
            pltpu.VMEM((_TW,), jnp.int32),
            pltpu.VMEM((_TW, _D), jnp.float32),
            pltpu.SemaphoreType.DMA,
        ],
    )
    def k(xi_hbm, p_hbm, xs_hbm, idx0_v, idx1_v, rows_v, sem):
        wid = lax.axis_index("s") * 2 + lax.axis_index("c")
        base = wid * _TW
        pltpu.sync_copy(p_hbm.at[wid, 0], idx0_v)
        pltpu.sync_copy(p_hbm.at[wid, 1], idx1_v)
        pltpu.sync_copy(xi_hbm.at[pl.ds(base, _TW)], rows_v)
        pltpu.async_copy(rows_v, xs_hbm.at[idx0_v], sem).wait()
        pltpu.async_copy(rows_v, xs_hbm.at[idx1_v], sem).wait()

    return k(xi, p_sc)


def _gmm_kernel(be_ref, xs_ref, wg_ref, wu_ref, wd_ref, es_ref,
                wg_p, wu_p, wd_p):
    b = pl.program_id(0)
    bf = jnp.bfloat16

    changed = jnp.logical_or(
        b == 0, be_ref[b] != be_ref[jnp.maximum(b - 1, 0)])

    @pl.when(changed)
    def _cast_panels():
        wg_p[...] = wg_ref[0].astype(bf)
        wu_p[...] = wu_ref[0].astype(bf)
        wd_p[...] = wd_ref[0].astype(bf)

    xs = xs_ref[...].astype(bf)
    hg = jnp.dot(xs, wg_p[...], preferred_element_type=jnp.float32)
    hu = jnp.dot(xs, wu_p[...], preferred_element_type=jnp.float32)
    inter = (jax.nn.silu(hg) * hu).astype(bf)
    eo = jnp.dot(inter, wd_p[...], preferred_element_type=jnp.float32)
    es_ref[...] = eo.astype(bf).astype(jnp.float32)


def _run_gmm(be, xs, w_gate, w_up, w_down):
    grid_spec = pltpu.PrefetchScalarGridSpec(
        num_scalar_prefetch=1,
        grid=(_NB,),
        in_specs=[
            pl.BlockSpec((_BR, _D), lambda b, be: (b, 0)),
            pl.BlockSpec((1, _D, _F), lambda b, be: (be[b], 0, 0)),
            pl.BlockSpec((1, _D, _F), lambda b, be: (be[b], 0, 0)),
            pl.BlockSpec((1, _F, _D), lambda b, be: (be[b], 0, 0)),
        ],
        out_specs=pl.BlockSpec((_BR, _D), lambda b, be: (b, 0)),
        scratch_shapes=[
            pltpu.VMEM((_D, _F), jnp.bfloat16),
            pltpu.VMEM((_D, _F), jnp.bfloat16),
            pltpu.VMEM((_F, _D), jnp.bfloat16),
        ],
    )
    return pl.pallas_call(
        _gmm_kernel,
        grid_spec=grid_spec,
        out_shape=jax.ShapeDtypeStruct((_PR, _D), jnp.float32),
        compiler_params=pltpu.CompilerParams(
            dimension_semantics=("arbitrary",)),
    )(be, xs, w_gate, w_up, w_down)


def _combine_gather(es, p0f, p1f):
    mesh = plsc.VectorSubcoreMesh(core_axis_name="c", subcore_axis_name="s")

    @functools.partial(
        pl.kernel, mesh=mesh,
        out_type=[jax.ShapeDtypeStruct((_T, _D), jnp.float32),
                  jax.ShapeDtypeStruct((_T, _D), jnp.float32)],
        scratch_types=[
            pltpu.VMEM((_TW,), jnp.int32),
            pltpu.VMEM((_TW, _D), jnp.float32),
            pltpu.SemaphoreType.DMA,
        ],
    )
    def k(es_hbm, p0_hbm, p1_hbm, g0_hbm, g1_hbm, idx_v, rows_v, sem):
        wid = lax.axis_index("s") * 2 + lax.axis_index("c")
        base = wid * _TW
        pltpu.sync_copy(p0_hbm.at[pl.ds(base, _TW)], idx_v)
        pltpu.async_copy(es_hbm.at[idx_v], rows_v, sem).wait()
        pltpu.sync_copy(rows_v, g0_hbm.at[pl.ds(base, _TW)])
        pltpu.sync_copy(p1_hbm.at[pl.ds(base, _TW)], idx_v)
        pltpu.async_copy(es_hbm.at[idx_v], rows_v, sem).wait()
        pltpu.sync_copy(rows_v, g1_hbm.at[pl.ds(base, _TW)])

    return k(es, p0f, p1f)


_BT = 512


def _final_kernel(x_ref, g0_ref, g1_ref, w_ref, sg_ref, su_ref, sd_ref,
                  out_ref):
    bf = jnp.bfloat16
    x = x_ref[...]
    hg = jnp.dot(x, sg_ref[...], preferred_element_type=jnp.float32)
    hu = jnp.dot(x, su_ref[...], preferred_element_type=jnp.float32)
    inter = (jax.nn.silu(hg) * hu).astype(bf)
    shared = jnp.dot(inter, sd_ref[...], preferred_element_type=jnp.float32)
    w = w_ref[...].astype(bf).astype(jnp.float32)
    out_ref[...] = (shared + w[:, 0:1] * g0_ref[...]
                    + w[:, 1:2] * g1_ref[...])


def _run_final(xbf, g0, g1, w, sh_gate, sh_up, sh_down):
    bf = jnp.bfloat16
    return pl.pallas_call(
        _final_kernel,
        grid=(_T // _BT,),
        in_specs=[
            pl.BlockSpec((_BT, _D), lambda i: (i, 0)),
            pl.BlockSpec((_BT, _D), lambda i: (i, 0)),
            pl.BlockSpec((_BT, _D), lambda i: (i, 0)),
            pl.BlockSpec((_BT, _K), lambda i: (i, 0)),
            pl.BlockSpec((_D, _F), lambda i: (0, 0)),
            pl.BlockSpec((_D, _F), lambda i: (0, 0)),
            pl.BlockSpec((_F, _D), lambda i: (0, 0)),
        ],
        out_specs=pl.BlockSpec((_BT, _D), lambda i: (i, 0)),
        out_shape=jax.ShapeDtypeStruct((_T, _D), jnp.float32),
    )(xbf, g0, g1, w, sh_gate.astype(bf), sh_up.astype(bf),
      sh_down.astype(bf))


def kernel(hidden_states, visual_token_mask, gate_w, corr_bias, w_gate, w_up,
           w_down, sh_gate, sh_up, sh_down):
    x = hidden_states.reshape(-1, _D)
    gate_wt = gate_w.T
    corr = corr_bias.reshape(1, _E)

    xbf, p, w, be2 = _run_router(x, gate_wt, corr)
    p_sc = p.reshape(_NW, _TW, _K).transpose(0, 2, 1)
    xs = _dispatch(x, p_sc)
    be = be2.reshape(_NB + 9)[:_NB]
    es = _run_gmm(be, xs, w_gate, w_up, w_down)
    p0f = p[:, 0].reshape(_T)
    p1f = p[:, 1].reshape(_T)
    g0, g1 = _combine_gather(es, p0f, p1f)
    out = _run_final(xbf, g0, g1, w, sh_gate, sh_up, sh_down)
    return out.reshape(hidden_states.shape)


# speedup vs baseline: 2.2786x; 2.2786x over previous
import jax
import jax.numpy as jnp
from jax.experimental import pallas as pl
from jax.experimental.pallas import tpu as pltpu

_T = 2048
_D = 1024
_E = 8
_K = 2
_F = 256
_BT = 512
_NE = _E + 1
_FC = _NE * _F


def _moe_block_kernel(x_ref, gate_wt_ref, corr_ref, wg_ref, wu_ref, wd_ref,
                      sg_ref, su_ref, sd_ref, out_ref,
                      wg_scr, wu_scr, wd_scr):
    i = pl.program_id(0)
    bf = jnp.bfloat16

    @pl.when(i == 0)
    def _build_panels():
        for e in range(_E):
            wg_scr[:, e * _F:(e + 1) * _F] = wg_ref[e].astype(bf)
            wu_scr[:, e * _F:(e + 1) * _F] = wu_ref[e].astype(bf)
            wd_scr[e * _F:(e + 1) * _F, :] = wd_ref[e].astype(bf)
        wg_scr[:, _E * _F:] = sg_ref[...].astype(bf)
        wu_scr[:, _E * _F:] = su_ref[...].astype(bf)
        wd_scr[_E * _F:, :] = sd_ref[...].astype(bf)

    x = x_ref[...].astype(bf)

    logits = jnp.dot(x, gate_wt_ref[...].astype(bf),
                     preferred_element_type=jnp.float32)
    hg = jnp.dot(x, wg_scr[...], preferred_element_type=jnp.float32)
    hu = jnp.dot(x, wu_scr[...], preferred_element_type=jnp.float32)

    scores = jax.nn.softmax(logits, axis=-1)
    biased = scores + corr_ref[...]

    eidx = jax.lax.broadcasted_iota(jnp.int32, (_BT, _E), 1)
    m1 = jnp.max(biased, axis=-1, keepdims=True)
    i1 = jnp.min(jnp.where(biased == m1, eidx, _E), axis=-1, keepdims=True)
    b2 = jnp.where(eidx == i1, -jnp.inf, biased)
    m2 = jnp.max(b2, axis=-1, keepdims=True)
    i2 = jnp.min(jnp.where(b2 == m2, eidx, _E), axis=-1, keepdims=True)
    w1 = jnp.sum(jnp.where(eidx == i1, scores, 0.0), axis=-1, keepdims=True)
    w2 = jnp.sum(jnp.where(eidx == i2, scores, 0.0), axis=-1, keepdims=True)
    denom = w1 + w2
    route = (jnp.where(eidx == i1, w1, 0.0)
             + jnp.where(eidx == i2, w2, 0.0)) / denom
    route = route.astype(bf).astype(jnp.float32)

    act = jax.nn.silu(hg) * hu
    parts = [act[:, e * _F:(e + 1) * _F] * route[:, e:e + 1]
             for e in range(_E)]
    parts.append(act[:, _E * _F:])
    inter = jnp.concatenate(parts, axis=1).astype(bf)
    out_ref[...] = jnp.dot(inter, wd_scr[...],
                           preferred_element_type=jnp.float32)


def kernel(hidden_states, visual_token_mask, gate_w, corr_bias, w_gate, w_up,
           w_down, sh_gate, sh_up, sh_down):
    x = hidden_states.reshape(-1, _D)
    gate_wt = gate_w.T
    corr = corr_bias.reshape(1, _E)

    grid = (_T // _BT,)
    out = pl.pallas_call(
        _moe_block_kernel,
        grid=grid,
        in_specs=[
            pl.BlockSpec((_BT, _D), lambda i: (i, 0)),
            pl.BlockSpec((_D, _E), lambda i: (0, 0)),
            pl.BlockSpec((1, _E), lambda i: (0, 0)),
            pl.BlockSpec((_E, _D, _F), lambda i: (0, 0, 0)),
            pl.BlockSpec((_E, _D, _F), lambda i: (0, 0, 0)),
            pl.BlockSpec((_E, _F, _D), lambda i: (0, 0, 0)),
            pl.BlockSpec((_D, _F), lambda i: (0, 0)),
            pl.BlockSpec((_D, _F), lambda i: (0, 0)),
            pl.BlockSpec((_F, _D), lambda i: (0, 0)),
        ],
        out_specs=pl.BlockSpec((_BT, _D), lambda i: (i, 0)),
        out_shape=jax.ShapeDtypeStruct((_T, _D), jnp.float32),
        scratch_shapes=[
            pltpu.VMEM((_D, _FC), jnp.bfloat16),
            pltpu.VMEM((_D, _FC), jnp.bfloat16),
            pltpu.VMEM((_FC, _D), jnp.bfloat16),
        ],
        compiler_params=pltpu.CompilerParams(
            dimension_semantics=("arbitrary",)),
    )(x, gate_wt, corr, w_gate, w_up, w_down, sh_gate, sh_up, sh_down)
    return out.reshape(hidden_states.shape)
